# baseline (speedup 1.0000x reference)
import functools
import math

import jax
import jax.numpy as jnp
from jax import lax
from jax.experimental import pallas as pl
from jax.experimental.pallas import tpu as pltpu
from jax.experimental.pallas import tpu_sc as plsc

RC = 5.0
NB = 8

_NUM_SC_CORES = 2
_NUM_SC_SUBCORES = 16
_NUM_WORKERS = _NUM_SC_CORES * _NUM_SC_SUBCORES


def _sc_gather(table, idx3, per_worker, chunk):
    nw, n_ch, chunk_ = idx3.shape
    assert chunk_ == chunk and n_ch * chunk == per_worker
    E = nw * per_worker
    D = table.shape[1]

    mesh = plsc.VectorSubcoreMesh(core_axis_name="c", subcore_axis_name="s")

    @functools.partial(
        pl.kernel,
        out_type=jax.ShapeDtypeStruct((E, D), table.dtype),
        mesh=mesh,
        scratch_types=[
            pltpu.VMEM((n_ch, chunk), jnp.int32),
            pltpu.VMEM((chunk, D), table.dtype),
            pltpu.VMEM((chunk, D), table.dtype),
            pltpu.SemaphoreType.DMA,
            pltpu.SemaphoreType.DMA,
        ],
    )
    def gather_kernel(table_hbm, idx_hbm, out_hbm, idx_v, buf0, buf1, sem0, sem1):
        wid = lax.axis_index("s") * _NUM_SC_CORES + lax.axis_index("c")
        base = wid * per_worker
        pltpu.sync_copy(idx_hbm.at[wid], idx_v)

        def start(j, buf, sem):
            pltpu.async_copy(table_hbm.at[idx_v.at[j]], buf, sem)

        def drain(j, buf, sem):
            pltpu.make_async_copy(table_hbm.at[idx_v.at[j]], buf, sem).wait()
            pltpu.sync_copy(buf, out_hbm.at[pl.ds(base + j * chunk, chunk)])

        start(0, buf0, sem0)

        def pair(i, _):
            j = 2 * i
            start(j + 1, buf1, sem1)
            drain(j, buf0, sem0)
            start(j + 2, buf0, sem0)
            drain(j + 1, buf1, sem1)
            return 0

        if n_ch % 2 == 0:
            lax.fori_loop(0, (n_ch - 2) // 2, pair, 0)
            start(n_ch - 1, buf1, sem1)
            drain(n_ch - 2, buf0, sem0)
            drain(n_ch - 1, buf1, sem1)
        else:
            lax.fori_loop(0, (n_ch - 1) // 2, pair, 0)
            drain(n_ch - 1, buf0, sem0)

    return gather_kernel(table, idx3)


def _tc_block_body(B, C, g_ref, e0_ref, e1_ref, d_ref, wr_ref, t3_ref,
                   w0_ref, w1_ref, wg_ref, bg_ref, *rest):
    out0_ref, out1_ref = rest[-2], rest[-1]
    f32 = jnp.float32
    dT = jnp.broadcast_to(d_ref[0:1, :], (NB, B))
    nn = lax.broadcasted_iota(jnp.int32, (NB, B), 0).astype(f32) + 1.0
    fc = 0.5 * (jnp.cos((math.pi / RC) * dT) + 1.0)
    fc = jnp.where(dT < RC, fc, 0.0)
    basis = (math.sqrt(2.0 / RC) * jnp.sin((math.pi / RC) * nn * dT) / dT) * fc
    tdot = (((0,), (0,)), ((), ()))
    rad = lax.dot_general(basis, wr_ref[...], tdot,
                          preferred_element_type=f32)
    U0 = rad[:, 0:C]
    U1 = rad[:, C:]
    urep = lax.dot_general(d_ref[1:4, :], t3_ref[...], tdot,
                           preferred_element_type=f32)
    g32 = g_ref[...]
    hi_mask = jnp.int32(-65536)

    def unlo(x):
        return lax.bitcast_convert_type(lax.shift_left(x, 16), f32)

    def unhi(x):
        return lax.bitcast_convert_type(lax.bitwise_and(x, hi_mask), f32)

    n0 = unlo(g32[:, 0:C])
    n1 = (unhi(g32[:, 0:C]), unlo(g32[:, C:]), unhi(g32[:, C:]))
    w0 = w0_ref[...]
    w1 = w1_ref[...]
    nu = jnp.zeros((B, C), f32)
    for k in range(3):
        nu = nu + n1[k] * urep[:, k * C:(k + 1) * C]
    m0 = n0 * U0 + nu * U1
    s0 = jnp.dot(m0, w0, preferred_element_type=f32)
    o0 = s0 * jax.nn.sigmoid(s0)
    out0_ref[...] = e0_ref[...] + o0
    f1 = n0 * U1
    s1 = []
    for k in range(3):
        m1k = f1 * urep[:, k * C:(k + 1) * C] + n1[k] * U0
        s1.append(jnp.dot(m1k, w1, preferred_element_type=f32))
    nrm = jnp.sqrt(s1[0] * s1[0] + s1[1] * s1[1] + s1[2] * s1[2] + 1e-8)
    gpre = jnp.dot(nrm, wg_ref[...], preferred_element_type=f32) + bg_ref[...]
    gt = gpre * jax.nn.sigmoid(gpre)
    for k in range(3):
        out1_ref[k, :, :] = e1_ref[k, :, :] + s1[k] * gt


def _tc_compute_slab(base, n_blocks, gathered_s, e0, e1p, du, W_rad, T3p,
                     W_si0, W_si1, Wg, bg2, B, prev):
    E, C = e0.shape

    def edge_spec(w, local=False):
        if local:
            return pl.BlockSpec((B, w), lambda i: (i, 0))
        return pl.BlockSpec((B, w), lambda i: (i + base, 0))

    def full_spec(shape):
        return pl.BlockSpec(shape, lambda i: (0,) * len(shape))

    in_specs = [
        edge_spec(2 * C, local=True),
        edge_spec(C),
        pl.BlockSpec((3, B, C), lambda i: (0, i + base, 0)),
        pl.BlockSpec((4, B), lambda i: (0, i + base)),
        full_spec(W_rad.shape),
        full_spec(T3p.shape),
        full_spec(W_si0.shape),
        full_spec(W_si1.shape),
        full_spec(Wg.shape),
        full_spec(bg2.shape),
    ]
    args = [gathered_s, e0, e1p, du, W_rad, T3p, W_si0, W_si1, Wg, bg2]
    aliases = {}
    if prev is not None:
        in_specs.append(pl.BlockSpec((8, C), lambda i: (0, 0)))
        in_specs.append(pl.BlockSpec((1, 8, C), lambda i: (0, 0, 0)))
        aliases = {len(args): 0, len(args) + 1: 1}
        args += [prev[0], prev[1]]

    out0, out1p = pl.pallas_call(
        functools.partial(_tc_block_body, B, C),
        grid=(n_blocks,),
        in_specs=in_specs,
        out_specs=[
            edge_spec(C),
            pl.BlockSpec((3, B, C), lambda i: (0, i + base, 0)),
        ],
        out_shape=[
            jax.ShapeDtypeStruct((E, C), jnp.float32),
            jax.ShapeDtypeStruct((3, E, C), jnp.float32),
        ],
        input_output_aliases=aliases,
        compiler_params=pltpu.CompilerParams(
            dimension_semantics=("arbitrary",),
        ),
    )(*args)
    return out0, out1p


def _largest_divisor(n, limit, mult):
    for b in range(limit - limit % mult, 0, -mult):
        if n % b == 0:
            return b
    return n


def kernel(node_0, node_1, edge_0, edge_1, dij, uij, W_rad, W_si0, W_si1,
           Wg, bg, edge_index):
    N, C = node_0.shape
    E = edge_0.shape[0]
    f32 = jnp.float32

    n1p = node_1.transpose(2, 0, 1)
    bf = jnp.bfloat16
    pairA = jnp.stack([node_0.astype(bf), n1p[0].astype(bf)], axis=-1)
    pairB = jnp.stack([n1p[1].astype(bf), n1p[2].astype(bf)], axis=-1)
    table = jnp.concatenate([
        lax.bitcast_convert_type(pairA, jnp.int32),
        lax.bitcast_convert_type(pairB, jnp.int32),
    ], axis=1)
    idx_j = edge_index[0].astype(jnp.int32)

    T3p = jnp.kron(jnp.eye(3, dtype=f32), jnp.ones((1, C), f32))

    unit = 128 * _NUM_WORKERS
    if E % (25 * 256) == 0 and E >= 25 * 256:
        u = E // 25
        sizes = [2 * u, 3 * u, 5 * u, 7 * u, 8 * u]
    elif E % 256 == 0 and E >= unit:
        units = E // unit
        rem = E - units * unit
        k = min(4, units)
        per, extra = divmod(units, k)
        sizes = [(per + (i < extra)) * unit for i in range(k)]
        sizes[-1] += rem
    else:
        sizes = [E]

    e1p = edge_1.transpose(2, 0, 1)
    du = jnp.concatenate([dij.reshape(1, E), uij.T], axis=0)
    bg2 = bg.reshape(1, C)
    row_bytes = table.shape[1] * table.dtype.itemsize
    outs = None
    base_e = 0
    for Es in sizes:
        per_worker = Es // _NUM_WORKERS
        chunk = 8
        for ch in range(8, 129, 8):
            if (per_worker % ch == 0
                    and 2 * ch * row_bytes + per_worker * 4 <= 480_000):
                chunk = ch
        idx3 = lax.dynamic_slice_in_dim(idx_j, base_e, Es).reshape(
            _NUM_WORKERS, per_worker // chunk, chunk)
        B = _largest_divisor(math.gcd(Es, base_e) if base_e else Es, 3328, 128)
        gathered_s = _sc_gather(table, idx3, per_worker, chunk)
        outs = _tc_compute_slab(
            base_e // B, Es // B, gathered_s, edge_0, e1p, du,
            W_rad, T3p, W_si0, W_si1, Wg, bg2, B, outs)
        base_e += Es
    out0, out1p = outs
    return (out0, out1p.transpose(1, 2, 0))


# speedup vs baseline: 1.0106x; 1.0106x over previous
import functools
import math

import jax
import jax.numpy as jnp
from jax import lax
from jax.experimental import pallas as pl
from jax.experimental.pallas import tpu as pltpu
from jax.experimental.pallas import tpu_sc as plsc

RC = 5.0
NB = 8

_NUM_SC_CORES = 2
_NUM_SC_SUBCORES = 16
_NUM_WORKERS = _NUM_SC_CORES * _NUM_SC_SUBCORES


def _sc_gather(table, idx3, per_worker, chunk):
    nw, n_ch, chunk_ = idx3.shape
    assert chunk_ == chunk and n_ch * chunk == per_worker
    E = nw * per_worker
    D = table.shape[1]

    mesh = plsc.VectorSubcoreMesh(core_axis_name="c", subcore_axis_name="s")

    @functools.partial(
        pl.kernel,
        out_type=jax.ShapeDtypeStruct((E, D), table.dtype),
        mesh=mesh,
        scratch_types=[
            pltpu.VMEM((n_ch, chunk), jnp.int32),
            pltpu.VMEM((chunk, D), table.dtype),
            pltpu.VMEM((chunk, D), table.dtype),
            pltpu.SemaphoreType.DMA,
            pltpu.SemaphoreType.DMA,
        ],
    )
    def gather_kernel(table_hbm, idx_hbm, out_hbm, idx_v, buf0, buf1, sem0, sem1):
        wid = lax.axis_index("s") * _NUM_SC_CORES + lax.axis_index("c")
        base = wid * per_worker
        pltpu.sync_copy(idx_hbm.at[wid], idx_v)

        def start(j, buf, sem):
            pltpu.async_copy(table_hbm.at[idx_v.at[j]], buf, sem)

        def drain(j, buf, sem):
            pltpu.make_async_copy(table_hbm.at[idx_v.at[j]], buf, sem).wait()
            pltpu.sync_copy(buf, out_hbm.at[pl.ds(base + j * chunk, chunk)])

        start(0, buf0, sem0)

        def pair(i, _):
            j = 2 * i
            start(j + 1, buf1, sem1)
            drain(j, buf0, sem0)
            start(j + 2, buf0, sem0)
            drain(j + 1, buf1, sem1)
            return 0

        if n_ch % 2 == 0:
            lax.fori_loop(0, (n_ch - 2) // 2, pair, 0)
            start(n_ch - 1, buf1, sem1)
            drain(n_ch - 2, buf0, sem0)
            drain(n_ch - 1, buf1, sem1)
        else:
            lax.fori_loop(0, (n_ch - 1) // 2, pair, 0)
            drain(n_ch - 1, buf0, sem0)

    return gather_kernel(table, idx3)


def _tc_block_body(B, C, g_ref, e0_ref, e1_ref, d_ref, u_ref, wr_ref, t3_ref,
                   w0_ref, w1_ref, wg_ref, bg_ref, *rest):
    out0_ref, out1_ref = rest[-2], rest[-1]
    f32 = jnp.float32
    dT = jnp.broadcast_to(d_ref[...], (NB, B))
    nn = lax.broadcasted_iota(jnp.int32, (NB, B), 0).astype(f32) + 1.0
    fc = 0.5 * (jnp.cos((math.pi / RC) * dT) + 1.0)
    fc = jnp.where(dT < RC, fc, 0.0)
    basis = (math.sqrt(2.0 / RC) * jnp.sin((math.pi / RC) * nn * dT) / dT) * fc
    tdot = (((0,), (0,)), ((), ()))
    rad = lax.dot_general(basis, wr_ref[...], tdot,
                          preferred_element_type=f32)
    U0 = rad[:, 0:C]
    U1 = rad[:, C:]
    urep = lax.dot_general(u_ref[...], t3_ref[...], tdot,
                           preferred_element_type=f32)
    g32 = g_ref[...]
    hi_mask = jnp.int32(-65536)

    def unlo(x):
        return lax.bitcast_convert_type(lax.shift_left(x, 16), f32)

    def unhi(x):
        return lax.bitcast_convert_type(lax.bitwise_and(x, hi_mask), f32)

    n0 = unlo(g32[:, 0:C])
    n1 = (unhi(g32[:, 0:C]), unlo(g32[:, C:]), unhi(g32[:, C:]))
    w0 = w0_ref[...]
    w1 = w1_ref[...]
    nu = jnp.zeros((B, C), f32)
    for k in range(3):
        nu = nu + n1[k] * urep[:, k * C:(k + 1) * C]
    m0 = n0 * U0 + nu * U1
    s0 = jnp.dot(m0, w0, preferred_element_type=f32)
    o0 = s0 * jax.nn.sigmoid(s0)
    out0_ref[...] = e0_ref[...] + o0
    f1 = n0 * U1
    s1 = []
    for k in range(3):
        m1k = f1 * urep[:, k * C:(k + 1) * C] + n1[k] * U0
        s1.append(jnp.dot(m1k, w1, preferred_element_type=f32))
    nrm = jnp.sqrt(s1[0] * s1[0] + s1[1] * s1[1] + s1[2] * s1[2] + 1e-8)
    gpre = jnp.dot(nrm, wg_ref[...], preferred_element_type=f32) + bg_ref[...]
    gt = gpre * jax.nn.sigmoid(gpre)
    for k in range(3):
        out1_ref[k, :, :] = e1_ref[k, :, :] + s1[k] * gt


def _tc_compute_slab(base, n_blocks, gathered_s, e0, e1p, d2, uT, W_rad, T3p,
                     W_si0, W_si1, Wg, bg2, B, prev):
    E, C = e0.shape

    def edge_spec(w, local=False):
        if local:
            return pl.BlockSpec((B, w), lambda i: (i, 0))
        return pl.BlockSpec((B, w), lambda i: (i + base, 0))

    def full_spec(shape):
        return pl.BlockSpec(shape, lambda i: (0,) * len(shape))

    in_specs = [
        edge_spec(2 * C, local=True),
        edge_spec(C),
        pl.BlockSpec((3, B, C), lambda i: (0, i + base, 0)),
        pl.BlockSpec((1, B), lambda i: (0, i + base)),
        pl.BlockSpec((3, B), lambda i: (0, i + base)),
        full_spec(W_rad.shape),
        full_spec(T3p.shape),
        full_spec(W_si0.shape),
        full_spec(W_si1.shape),
        full_spec(Wg.shape),
        full_spec(bg2.shape),
    ]
    args = [gathered_s, e0, e1p, d2, uT, W_rad, T3p, W_si0, W_si1, Wg, bg2]
    aliases = {}
    if prev is not None:
        in_specs.append(pl.BlockSpec((8, C), lambda i: (0, 0)))
        in_specs.append(pl.BlockSpec((1, 8, C), lambda i: (0, 0, 0)))
        aliases = {len(args): 0, len(args) + 1: 1}
        args += [prev[0], prev[1]]

    out0, out1p = pl.pallas_call(
        functools.partial(_tc_block_body, B, C),
        grid=(n_blocks,),
        in_specs=in_specs,
        out_specs=[
            edge_spec(C),
            pl.BlockSpec((3, B, C), lambda i: (0, i + base, 0)),
        ],
        out_shape=[
            jax.ShapeDtypeStruct((E, C), jnp.float32),
            jax.ShapeDtypeStruct((3, E, C), jnp.float32),
        ],
        input_output_aliases=aliases,
        compiler_params=pltpu.CompilerParams(
            dimension_semantics=("arbitrary",),
        ),
    )(*args)
    return out0, out1p


def _largest_divisor(n, limit, mult):
    for b in range(limit - limit % mult, 0, -mult):
        if n % b == 0:
            return b
    return n


def kernel(node_0, node_1, edge_0, edge_1, dij, uij, W_rad, W_si0, W_si1,
           Wg, bg, edge_index):
    N, C = node_0.shape
    E = edge_0.shape[0]
    f32 = jnp.float32

    n1p = node_1.transpose(2, 0, 1)
    bf = jnp.bfloat16
    pairA = jnp.stack([node_0.astype(bf), n1p[0].astype(bf)], axis=-1)
    pairB = jnp.stack([n1p[1].astype(bf), n1p[2].astype(bf)], axis=-1)
    table = jnp.concatenate([
        lax.bitcast_convert_type(pairA, jnp.int32),
        lax.bitcast_convert_type(pairB, jnp.int32),
    ], axis=1)
    idx_j = edge_index[0].astype(jnp.int32)

    T3p = jnp.kron(jnp.eye(3, dtype=f32), jnp.ones((1, C), f32))

    unit = 128 * _NUM_WORKERS
    if E % (25 * 256) == 0 and E >= 25 * 256:
        u = E // 25
        sizes = [2 * u, 3 * u, 5 * u, 7 * u, 8 * u]
    elif E % 256 == 0 and E >= unit:
        units = E // unit
        rem = E - units * unit
        k = min(4, units)
        per, extra = divmod(units, k)
        sizes = [(per + (i < extra)) * unit for i in range(k)]
        sizes[-1] += rem
    else:
        sizes = [E]

    e1p = edge_1.transpose(2, 0, 1)
    d2 = dij.reshape(1, E)
    uT = uij.T
    bg2 = bg.reshape(1, C)
    row_bytes = table.shape[1] * table.dtype.itemsize
    outs = None
    base_e = 0
    for Es in sizes:
        per_worker = Es // _NUM_WORKERS
        chunk = 8
        for ch in range(8, 129, 8):
            if (per_worker % ch == 0
                    and 2 * ch * row_bytes + per_worker * 4 <= 480_000):
                chunk = ch
        idx3 = lax.dynamic_slice_in_dim(idx_j, base_e, Es).reshape(
            _NUM_WORKERS, per_worker // chunk, chunk)
        B = _largest_divisor(math.gcd(Es, base_e) if base_e else Es, 3328, 128)
        gathered_s = _sc_gather(table, idx3, per_worker, chunk)
        outs = _tc_compute_slab(
            base_e // B, Es // B, gathered_s, edge_0, e1p, d2, uT,
            W_rad, T3p, W_si0, W_si1, Wg, bg2, B, outs)
        base_e += Es
    out0, out1p = outs
    return (out0, out1p.transpose(1, 2, 0))
